# baseline (speedup 1.0000x reference)
import functools

import jax
import jax.numpy as jnp
from jax import lax
from jax.experimental import pallas as pl
from jax.experimental.pallas import tpu as pltpu
from jax.experimental.pallas import tpu_sc as plsc

VOCAB = 100000
EMB = 64
HID = 128
CTX = 200


_HPAD = 208


def _sc_gather_sum(idx_pad, table):
    mesh = plsc.VectorSubcoreMesh(core_axis_name="c", subcore_axis_name="s")

    @functools.partial(
        pl.kernel,
        mesh=mesh,
        out_type=jax.ShapeDtypeStruct((EMB,), jnp.float32),
        scratch_types=[
            pltpu.VMEM((_HPAD,), jnp.int32),
            pltpu.VMEM((CTX, EMB), jnp.float32),
            pltpu.VMEM((EMB,), jnp.float32),
            pltpu.SemaphoreType.DMA,
        ],
    )
    def k(idx_hbm, table_hbm, out_hbm, idx_v, rows_v, acc_v, sem):
        wid = lax.axis_index("s") * 2 + lax.axis_index("c")

        @pl.when(wid == 0)
        def _():
            pltpu.sync_copy(idx_hbm, idx_v)
            pltpu.make_async_copy(
                table_hbm.at[pl.ds(0, 1)], rows_v.at[pl.ds(0, 1)], sem
            ).start()
            pltpu.make_async_copy(
                table_hbm.at[pl.ds(0, 1)], rows_v.at[pl.ds(0, 1)], sem
            ).wait()
            acc_v[pl.ds(0, 16)] = rows_v[0, pl.ds(0, 16)]
            acc_v[pl.ds(16, 16)] = rows_v[0, pl.ds(16, 16)]
            acc_v[pl.ds(32, 16)] = rows_v[0, pl.ds(32, 16)]
            acc_v[pl.ds(48, 16)] = rows_v[0, pl.ds(48, 16)]
            pltpu.sync_copy(acc_v, out_hbm)

    return k(idx_pad, table)


_TILE = 12544
_NT = (VOCAB + _TILE - 1) // _TILE
_PADV = _NT * _TILE


def _tc_body(e_ref, w1_ref, b1_ref, w2_ref, b2_ref, out_ref, m_ref, s_ref):
    i = pl.program_id(0)

    @pl.when(i == 0)
    def _():
        m_ref[0] = -jnp.inf
        s_ref[0] = 0.0

    h = lax.dot_general(
        e_ref[...], w1_ref[...],
        dimension_numbers=(((1,), (1,)), ((), ())),
        preferred_element_type=jnp.float32,
    ) + b1_ref[...]
    h = jnp.maximum(h, 0.0)

    logits = lax.dot_general(
        h, w2_ref[...],
        dimension_numbers=(((1,), (1,)), ((), ())),
        preferred_element_type=jnp.float32,
    ) + b2_ref[...]

    col = i * _TILE + lax.broadcasted_iota(jnp.int32, (1, _TILE), 1)
    masked = jnp.where(col < VOCAB, logits, -jnp.inf)

    m_old = m_ref[0]
    m_new = jnp.maximum(m_old, jnp.max(masked))
    s_ref[0] = s_ref[0] * jnp.exp(m_old - m_new) + jnp.sum(jnp.exp(masked - m_new))
    m_ref[0] = m_new

    out_ref[:, pl.ds(i * _TILE, _TILE)] = logits

    @pl.when(i == _NT - 1)
    def _():
        lse = m_ref[0] + jnp.log(s_ref[0])
        out_ref[...] = out_ref[...] - lse


def _tc_forward(e, W1, b1_2d, W2, b2_2d):
    return pl.pallas_call(
        _tc_body,
        grid=(_NT,),
        in_specs=[
            pl.BlockSpec((1, EMB), lambda i: (0, 0)),
            pl.BlockSpec((HID, EMB), lambda i: (0, 0)),
            pl.BlockSpec((1, HID), lambda i: (0, 0)),
            pl.BlockSpec((_TILE, HID), lambda i: (i, 0)),
            pl.BlockSpec((1, _TILE), lambda i: (0, i)),
        ],
        out_specs=pl.BlockSpec((1, _PADV), lambda i: (0, 0)),
        out_shape=jax.ShapeDtypeStruct((1, _PADV), jnp.float32),
        scratch_shapes=[
            pltpu.SMEM((1,), jnp.float32),
            pltpu.SMEM((1,), jnp.float32),
        ],
    )(e, W1, b1_2d, W2, b2_2d)


def kernel(inputs, table, W1, b1, W2, b2):
    idx = inputs.astype(jnp.int32)
    idx_pad = jnp.pad(idx, (0, _HPAD - CTX))
    e = _sc_gather_sum(idx_pad, table).reshape(1, EMB)
    out = _tc_forward(e, W1, b1.reshape(1, HID), W2, b2.reshape(1, VOCAB))
    return out[:, :VOCAB]


# speedup vs baseline: 1.2095x; 1.2095x over previous
import jax
import jax.numpy as jnp
from jax import lax
from jax.experimental import pallas as pl
from jax.experimental.pallas import tpu as pltpu

VOCAB = 100000
EMB = 64
HID = 128
CTX = 200

_TILE = 12544
_NT = (VOCAB + _TILE - 1) // _TILE
_PADV = _NT * _TILE


def _body(idx_ref, table_ref, w1_ref, b1_ref, w2_ref, b2_ref, out_ref,
          rows, hsc, m_ref, s_ref, sem):
    i = pl.program_id(0)

    @pl.when(i == 0)
    def _():
        m_ref[0] = -jnp.inf
        s_ref[0] = 0.0
        copies = [
            pltpu.make_async_copy(
                table_ref.at[pl.ds(idx_ref[j], 1)],
                rows.at[pl.ds(j, 1)],
                sem,
            )
            for j in range(CTX)
        ]
        for c in copies:
            c.start()
        for c in copies:
            c.wait()
        e = jnp.sum(rows[...], axis=0, keepdims=True)
        h = lax.dot_general(
            e, w1_ref[...],
            dimension_numbers=(((1,), (1,)), ((), ())),
            preferred_element_type=jnp.float32,
        ) + b1_ref[...]
        hsc[...] = jnp.maximum(h, 0.0)

    logits = lax.dot_general(
        hsc[...], w2_ref[...],
        dimension_numbers=(((1,), (1,)), ((), ())),
        preferred_element_type=jnp.float32,
    ) + b2_ref[...]

    col = i * _TILE + lax.broadcasted_iota(jnp.int32, (1, _TILE), 1)
    masked = jnp.where(col < VOCAB, logits, -jnp.inf)

    m_old = m_ref[0]
    m_new = jnp.maximum(m_old, jnp.max(masked))
    s_ref[0] = s_ref[0] * jnp.exp(m_old - m_new) + jnp.sum(jnp.exp(masked - m_new))
    m_ref[0] = m_new

    out_ref[:, pl.ds(i * _TILE, _TILE)] = logits

    @pl.when(i == _NT - 1)
    def _():
        lse = m_ref[0] + jnp.log(s_ref[0])
        out_ref[...] = out_ref[...] - lse


def kernel(inputs, table, W1, b1, W2, b2):
    idx = inputs.astype(jnp.int32)
    out = pl.pallas_call(
        _body,
        grid=(_NT,),
        in_specs=[
            pl.BlockSpec(memory_space=pltpu.SMEM),
            pl.BlockSpec(memory_space=pl.ANY),
            pl.BlockSpec((HID, EMB), lambda i: (0, 0)),
            pl.BlockSpec((1, HID), lambda i: (0, 0)),
            pl.BlockSpec((_TILE, HID), lambda i: (i, 0)),
            pl.BlockSpec((1, _TILE), lambda i: (0, i)),
        ],
        out_specs=pl.BlockSpec((1, _PADV), lambda i: (0, 0)),
        out_shape=jax.ShapeDtypeStruct((1, _PADV), jnp.float32),
        scratch_shapes=[
            pltpu.VMEM((CTX, EMB), jnp.float32),
            pltpu.VMEM((1, HID), jnp.float32),
            pltpu.SMEM((1,), jnp.float32),
            pltpu.SMEM((1,), jnp.float32),
            pltpu.SemaphoreType.DMA,
        ],
    )(idx, table, W1, b1.reshape(1, HID), W2, b2.reshape(1, VOCAB))
    return out[:, :VOCAB]
